# Initial kernel scaffold; baseline (speedup 1.0000x reference)
import jax
import jax.numpy as jnp
from jax.experimental import pallas as pl


def kernel(inputs_left, inputs_right, emb, W1, b1, W2, b2):
    raise NotImplementedError("write your pallas kernel here")


# speedup vs baseline: 1.5785x; 1.5785x over previous
import functools

import jax
import jax.numpy as jnp
from jax import lax
from jax.experimental import pallas as pl
from jax.experimental.pallas import tpu as pltpu
from jax.experimental.pallas import tpu_sc as plsc

EMB = 128
CTX = 200
H1 = 1024
OUT = 128

_NC, _NS = 2, 16
_NW = _NC * _NS
_B_PAD = 512
_B_PER_W = _B_PAD // _NW

_KB = 1280
_NK = (CTX * EMB) // _KB


def _gather_body(table_hbm, idx_hbm, out_hbm, idx_v, rows_v, sem):
    wid = lax.axis_index("s") * _NC + lax.axis_index("c")
    base = wid * _B_PER_W
    pltpu.sync_copy(idx_hbm.at[pl.ds(base, _B_PER_W)], idx_v)
    pltpu.async_copy(table_hbm.at[idx_v], rows_v, sem).wait()
    pltpu.sync_copy(rows_v, out_hbm.at[pl.ds(base, _B_PER_W)])


@functools.cache
def _make_gather():
    return functools.partial(
        pl.kernel,
        mesh=plsc.VectorSubcoreMesh(core_axis_name="c", subcore_axis_name="s"),
        out_type=jax.ShapeDtypeStruct((_B_PAD, EMB), jnp.float32),
        scratch_types=[
            pltpu.VMEM((_B_PER_W,), jnp.int32),
            pltpu.VMEM((_B_PER_W, EMB), jnp.float32),
            pltpu.SemaphoreType.DMA,
        ],
    )(_gather_body)


def _mlp_body(x_ref, w1_ref, b1_ref, w2_ref, b2_ref, out_ref, acc_ref):
    k = pl.program_id(0)

    @pl.when(k == 0)
    def _():
        acc_ref[...] = jnp.zeros_like(acc_ref)

    acc_ref[...] += lax.dot_general(
        x_ref[...], w1_ref[...],
        (((1,), (1,)), ((), ())),
        preferred_element_type=jnp.float32)

    @pl.when(k == _NK - 1)
    def _():
        h = jnp.maximum(acc_ref[...] + b1_ref[...], 0.0)
        o = lax.dot_general(
            h, w2_ref[...],
            (((1,), (1,)), ((), ())),
            preferred_element_type=jnp.float32)
        out_ref[...] = jnp.maximum(o + b2_ref[...], 0.0)


def _mlp(x, W1, b1, W2, b2):
    return pl.pallas_call(
        _mlp_body,
        grid=(_NK,),
        in_specs=[
            pl.BlockSpec((2, _KB), lambda k: (0, k)),
            pl.BlockSpec((H1, _KB), lambda k: (0, k)),
            pl.BlockSpec((1, H1), lambda k: (0, 0)),
            pl.BlockSpec((OUT, H1), lambda k: (0, 0)),
            pl.BlockSpec((1, OUT), lambda k: (0, 0)),
        ],
        out_specs=pl.BlockSpec((2, OUT), lambda k: (0, 0)),
        out_shape=jax.ShapeDtypeStruct((2, OUT), jnp.float32),
        scratch_shapes=[pltpu.VMEM((2, H1), jnp.float32)],
    )(x, W1, b1, W2, b2)


def kernel(inputs_left, inputs_right, emb, W1, b1, W2, b2):
    idx = jnp.concatenate([
        inputs_left.astype(jnp.int32),
        inputs_right.astype(jnp.int32),
        jnp.zeros((_B_PAD - 2 * CTX,), jnp.int32),
    ])
    rows = _make_gather()(emb, idx)
    x = rows[: 2 * CTX].reshape(2, CTX * EMB)
    out = _mlp(x, W1, b1.reshape(1, H1), W2, b2.reshape(1, OUT))
    return (out[0:1], out[1:2])


# speedup vs baseline: 1.6331x; 1.0346x over previous
import functools

import jax
import jax.numpy as jnp
from jax import lax
from jax.experimental import pallas as pl
from jax.experimental.pallas import tpu as pltpu
from jax.experimental.pallas import tpu_sc as plsc

EMB = 128
CTX = 200
H1 = 1024
OUT = 128

_NC, _NS = 2, 16
_NW = _NC * _NS
_B_PAD = 512
_B_PER_W = _B_PAD // _NW

_HB = 128
_NH = H1 // _HB


def _gather_body(table_hbm, idx_hbm, out_hbm, idx_v, rows_v, sem):
    wid = lax.axis_index("s") * _NC + lax.axis_index("c")
    base = wid * _B_PER_W
    pltpu.sync_copy(idx_hbm.at[pl.ds(base, _B_PER_W)], idx_v)
    pltpu.async_copy(table_hbm.at[idx_v], rows_v, sem).wait()
    pltpu.sync_copy(rows_v, out_hbm.at[pl.ds(base, _B_PER_W)])


@functools.cache
def _make_gather():
    return functools.partial(
        pl.kernel,
        mesh=plsc.VectorSubcoreMesh(core_axis_name="c", subcore_axis_name="s"),
        out_type=jax.ShapeDtypeStruct((_B_PAD, EMB), jnp.float32),
        scratch_types=[
            pltpu.VMEM((_B_PER_W,), jnp.int32),
            pltpu.VMEM((_B_PER_W, EMB), jnp.float32),
            pltpu.SemaphoreType.DMA,
        ],
    )(_gather_body)


def _mlp_body(x_ref, w1_ref, b1_ref, w2_ref, b2_ref, out_ref, h_ref):
    k = pl.program_id(0)
    xs = lax.dot_general(
        x_ref[...], w1_ref[...],
        (((1,), (1,)), ((), ())),
        preferred_element_type=jnp.float32)
    h_ref[:, pl.ds(k * _HB, _HB)] = jnp.maximum(xs + b1_ref[...], 0.0)

    @pl.when(k == _NH - 1)
    def _():
        o = lax.dot_general(
            h_ref[...], w2_ref[...],
            (((1,), (1,)), ((), ())),
            preferred_element_type=jnp.float32)
        out_ref[...] = jnp.maximum(o + b2_ref[...], 0.0)


def _mlp(x, W1, b1, W2, b2):
    return pl.pallas_call(
        _mlp_body,
        grid=(_NH,),
        in_specs=[
            pl.BlockSpec((2, CTX * EMB), lambda k: (0, 0)),
            pl.BlockSpec((_HB, CTX * EMB), lambda k: (k, 0)),
            pl.BlockSpec((1, _HB), lambda k: (0, k)),
            pl.BlockSpec((OUT, H1), lambda k: (0, 0)),
            pl.BlockSpec((1, OUT), lambda k: (0, 0)),
        ],
        out_specs=pl.BlockSpec((2, OUT), lambda k: (0, 0)),
        out_shape=jax.ShapeDtypeStruct((2, OUT), jnp.float32),
        scratch_shapes=[pltpu.VMEM((2, H1), jnp.float32)],
    )(x, W1, b1, W2, b2)


def kernel(inputs_left, inputs_right, emb, W1, b1, W2, b2):
    idx = jnp.concatenate([
        inputs_left.astype(jnp.int32),
        inputs_right.astype(jnp.int32),
        jnp.zeros((_B_PAD - 2 * CTX,), jnp.int32),
    ])
    rows = _make_gather()(emb, idx)
    x = rows[: 2 * CTX].reshape(2, CTX * EMB)
    out = _mlp(x, W1, b1.reshape(1, H1), W2, b2.reshape(1, OUT))
    return (out[0:1], out[1:2])


# speedup vs baseline: 1.6444x; 1.0070x over previous
import functools

import jax
import jax.numpy as jnp
from jax import lax
from jax.experimental import pallas as pl
from jax.experimental.pallas import tpu as pltpu
from jax.experimental.pallas import tpu_sc as plsc

EMB = 128
CTX = 200
H1 = 1024
OUT = 128

_NC, _NS = 2, 16
_NW = _NC * _NS
_B_PAD = 512
_B_PER_W = _B_PAD // _NW

_HB = 128
_NH = (H1 // 2) // _HB


def _gather_body(table_hbm, idx_hbm, out_hbm, idx_v, rows_v, sem):
    wid = lax.axis_index("s") * _NC + lax.axis_index("c")
    base = wid * _B_PER_W
    pltpu.sync_copy(idx_hbm.at[pl.ds(base, _B_PER_W)], idx_v)
    pltpu.async_copy(table_hbm.at[idx_v], rows_v, sem).wait()
    pltpu.sync_copy(rows_v, out_hbm.at[pl.ds(base, _B_PER_W)])


@functools.cache
def _make_gather():
    return functools.partial(
        pl.kernel,
        mesh=plsc.VectorSubcoreMesh(core_axis_name="c", subcore_axis_name="s"),
        out_type=jax.ShapeDtypeStruct((_B_PAD, EMB), jnp.float32),
        scratch_types=[
            pltpu.VMEM((_B_PER_W,), jnp.int32),
            pltpu.VMEM((_B_PER_W, EMB), jnp.float32),
            pltpu.SemaphoreType.DMA,
        ],
    )(_gather_body)


def _mlp_body(x_ref, w1a_ref, w1b_ref, b1_ref, w2_ref, b2_ref,
              outl_ref, outr_ref, h_ref):
    k = pl.program_id(0)
    xa = lax.dot_general(
        x_ref[...], w1a_ref[...],
        (((1,), (1,)), ((), ())),
        preferred_element_type=jnp.float32)
    h_ref[:, pl.ds(k * _HB, _HB)] = jnp.maximum(
        xa + b1_ref[:, pl.ds(k * _HB, _HB)], 0.0)
    xb = lax.dot_general(
        x_ref[...], w1b_ref[...],
        (((1,), (1,)), ((), ())),
        preferred_element_type=jnp.float32)
    h_ref[:, pl.ds(H1 // 2 + k * _HB, _HB)] = jnp.maximum(
        xb + b1_ref[:, pl.ds(H1 // 2 + k * _HB, _HB)], 0.0)

    @pl.when(k == _NH - 1)
    def _():
        o = lax.dot_general(
            h_ref[...], w2_ref[...],
            (((1,), (1,)), ((), ())),
            preferred_element_type=jnp.float32)
        o = jnp.maximum(o + b2_ref[...], 0.0)
        outl_ref[...] = o[0:1, :]
        outr_ref[...] = o[1:2, :]


def _mlp(x, W1, b1, W2, b2):
    return pl.pallas_call(
        _mlp_body,
        grid=(_NH,),
        in_specs=[
            pl.BlockSpec((2, CTX * EMB), lambda k: (0, 0)),
            pl.BlockSpec((_HB, CTX * EMB), lambda k: (k, 0)),
            pl.BlockSpec((_HB, CTX * EMB), lambda k: (k + _NH, 0)),
            pl.BlockSpec((1, H1), lambda k: (0, 0)),
            pl.BlockSpec((OUT, H1), lambda k: (0, 0)),
            pl.BlockSpec((1, OUT), lambda k: (0, 0)),
        ],
        out_specs=[
            pl.BlockSpec((1, OUT), lambda k: (0, 0)),
            pl.BlockSpec((1, OUT), lambda k: (0, 0)),
        ],
        out_shape=[
            jax.ShapeDtypeStruct((1, OUT), jnp.float32),
            jax.ShapeDtypeStruct((1, OUT), jnp.float32),
        ],
        scratch_shapes=[pltpu.VMEM((2, H1), jnp.float32)],
    )(x, W1, W1, b1, W2, b2)


def kernel(inputs_left, inputs_right, emb, W1, b1, W2, b2):
    idx = jnp.concatenate([
        inputs_left.astype(jnp.int32),
        inputs_right.astype(jnp.int32),
        jnp.zeros((_B_PAD - 2 * CTX,), jnp.int32),
    ])
    rows = _make_gather()(emb, idx)
    x = rows[: 2 * CTX].reshape(2, CTX * EMB)
    out_l, out_r = _mlp(x, W1, b1.reshape(1, H1), W2, b2.reshape(1, OUT))
    return (out_l, out_r)
